# baseline (speedup 1.0000x reference)
import functools

import jax
import jax.numpy as jnp
from jax.experimental import pallas as pl
from jax.experimental.pallas import tpu as pltpu

NUM_EXPERTS = 8
DIM = 1024
HIDDEN_DIM = 4096
T = 32
H_BLK = 2048
N_HBLK = HIDDEN_DIM // H_BLK
N_CORES = 2
E_PER_CORE = NUM_EXPERTS // N_CORES


def _moe_kernel(idx_ref, x_ref, fc1_ref, fc2_ref, out_ref):
    c = pl.program_id(0)
    ei = pl.program_id(1)
    hb = pl.program_id(2)
    e = c * E_PER_CORE + ei

    @pl.when(jnp.logical_and(ei == 0, hb == 0))
    def _init():
        out_ref[...] = jnp.zeros_like(out_ref)

    mask = idx_ref[...] == e
    xm = jnp.where(mask, x_ref[...], 0.0)
    h = jax.lax.dot_general(
        xm, fc1_ref[0],
        dimension_numbers=(((1,), (1,)), ((), ())),
        preferred_element_type=jnp.float32,
    )
    h = h * jax.nn.sigmoid(h)
    out_ref[...] += jax.lax.dot_general(
        h, fc2_ref[0],
        dimension_numbers=(((1,), (1,)), ((), ())),
        preferred_element_type=jnp.float32,
    )


@jax.jit
def kernel(x, expert_idx, fc1_weight, fc2_weight):
    idx2d = expert_idx.astype(jnp.int32).reshape(T, 1)
    grid = (N_CORES, E_PER_CORE, N_HBLK)
    partial = pl.pallas_call(
        _moe_kernel,
        grid=grid,
        in_specs=[
            pl.BlockSpec((T, 1), lambda c, ei, hb: (0, 0)),
            pl.BlockSpec((T, DIM), lambda c, ei, hb: (0, 0)),
            pl.BlockSpec((1, H_BLK, DIM),
                         lambda c, ei, hb: (c * E_PER_CORE + ei, hb, 0)),
            pl.BlockSpec((1, DIM, H_BLK),
                         lambda c, ei, hb: (c * E_PER_CORE + ei, 0, hb)),
        ],
        out_specs=pl.BlockSpec((1, T, DIM), lambda c, ei, hb: (c, 0, 0)),
        out_shape=jax.ShapeDtypeStruct((N_CORES, T, DIM), jnp.float32),
        compiler_params=pltpu.CompilerParams(
            dimension_semantics=("parallel", "arbitrary", "arbitrary"),
        ),
    )(idx2d, x, fc1_weight, fc2_weight)
    return partial[0] + partial[1]
